# baseline (speedup 1.0000x reference)
import functools

import jax
import jax.numpy as jnp
from jax import lax
from jax.experimental import pallas as pl
from jax.experimental.pallas import tpu as pltpu
from jax.experimental.pallas import tpu_sc as plsc

_NUM_CORES = 2
_NUM_SUBCORES = 16
_NUM_WORKERS = _NUM_CORES * _NUM_SUBCORES
_CHUNK = 80
_NBUF = 4
_NSPLIT = 4
_BB = 8


def _make_sc_gather(n_rows: int, d: int):
    rows_per_worker = n_rows // _NUM_WORKERS
    k_per_worker = rows_per_worker // _CHUNK
    n_steps = k_per_worker // _NBUF
    mesh = plsc.VectorSubcoreMesh(
        core_axis_name="c",
        subcore_axis_name="s",
        num_cores=_NUM_CORES,
        num_subcores=_NUM_SUBCORES,
    )

    @functools.partial(
        pl.kernel,
        out_type=jax.ShapeDtypeStruct((n_rows, d), jnp.float32),
        mesh=mesh,
        scratch_types=[
            pltpu.VMEM((rows_per_worker,), jnp.int32),
            pltpu.VMEM((_NBUF, _CHUNK, d), jnp.float32),
            pltpu.SemaphoreType.DMA((_NBUF,)),
            pltpu.SemaphoreType.DMA((_NBUF,)),
        ],
    )
    def gather_kernel(idx_hbm, table_hbm, out_hbm, idx_v, bufs, gsem, ssem):
        wid = lax.axis_index("s") * _NUM_CORES + lax.axis_index("c")
        row0 = wid * rows_per_worker
        pltpu.sync_copy(idx_hbm.at[pl.ds(row0, rows_per_worker)], idx_v)

        def step(g, carry):
            sbase = g * (_NBUF * _CHUNK)
            for b in range(_NBUF):
                off = sbase + b * _CHUNK

                @pl.when(g > 0)
                def _():
                    pltpu.make_async_copy(
                        bufs.at[b], out_hbm.at[pl.ds(row0, _CHUNK)], ssem.at[b]
                    ).wait()

                pltpu.make_async_copy(
                    table_hbm.at[idx_v.at[pl.ds(off, _CHUNK)]],
                    bufs.at[b],
                    gsem.at[b],
                ).start()
            for b in range(_NBUF):
                off = sbase + b * _CHUNK
                pltpu.make_async_copy(
                    table_hbm.at[idx_v.at[pl.ds(off, _CHUNK)]],
                    bufs.at[b],
                    gsem.at[b],
                ).wait()
                pltpu.make_async_copy(
                    bufs.at[b], out_hbm.at[pl.ds(row0 + off, _CHUNK)], ssem.at[b]
                ).start()
            return carry

        lax.fori_loop(0, n_steps, step, 0)
        for b in range(_NBUF):
            pltpu.make_async_copy(
                bufs.at[b], out_hbm.at[pl.ds(row0, _CHUNK)], ssem.at[b]
            ).wait()

    return gather_kernel


def _retile_first(rows_flat, batch, hist, d, bs):
    def body(in_ref, out_ref):
        for j in range(_BB):
            out_ref[j] = in_ref[pl.ds(j * hist, hist), :]

    return pl.pallas_call(
        body,
        grid=(bs // _BB,),
        in_specs=[pl.BlockSpec((_BB * hist, d), lambda i: (i, 0))],
        out_specs=pl.BlockSpec((_BB, hist, d), lambda i: (i, 0, 0)),
        out_shape=jax.ShapeDtypeStruct((batch, hist, d), jnp.float32),
    )(rows_flat)


def _retile_next(out_prev, rows_flat, batch, hist, d, bs, slice_idx):
    nblk = bs // _BB
    base = slice_idx * nblk

    def body(in_ref, prev_ref, out_ref):
        del prev_ref
        for j in range(_BB):
            out_ref[j] = in_ref[pl.ds(j * hist, hist), :]

    return pl.pallas_call(
        body,
        grid=(nblk,),
        in_specs=[
            pl.BlockSpec((_BB * hist, d), lambda i: (i, 0)),
            pl.BlockSpec(memory_space=pl.ANY),
        ],
        out_specs=pl.BlockSpec((_BB, hist, d), lambda i, b=base: (b + i, 0, 0)),
        out_shape=jax.ShapeDtypeStruct((batch, hist, d), jnp.float32),
        input_output_aliases={1: 0},
    )(rows_flat, out_prev)


def kernel(input, table):
    batch, hist = input.shape
    v, d = table.shape
    bs = batch // _NSPLIT
    n_rows = bs * hist
    assert n_rows % (_NUM_WORKERS * _CHUNK * _NBUF) == 0
    assert bs % _BB == 0
    idx = input.reshape(batch * hist).astype(jnp.int32)
    gather = _make_sc_gather(n_rows, d)
    out = None
    for i in range(_NSPLIT):
        piece = gather(idx[i * n_rows : (i + 1) * n_rows], table)
        if out is None:
            out = _retile_first(piece, batch, hist, d, bs)
        else:
            out = _retile_next(out, piece, batch, hist, d, bs, i)
    return out


# speedup vs baseline: 2.6634x; 2.6634x over previous
import functools

import jax
import jax.numpy as jnp
from jax import lax
from jax.experimental import pallas as pl
from jax.experimental.pallas import tpu as pltpu
from jax.experimental.pallas import tpu_sc as plsc

_NUM_CORES = 2
_NUM_SUBCORES = 16
_NUM_WORKERS = _NUM_CORES * _NUM_SUBCORES
_NB = 2
_NBUF = 8


def _make_gather(batch: int, hist: int, d: int):
    bp_worker = batch // _NUM_WORKERS
    n_steps = bp_worker // (_NBUF * _NB)
    mesh = plsc.VectorSubcoreMesh(
        core_axis_name="c",
        subcore_axis_name="s",
        num_cores=_NUM_CORES,
        num_subcores=_NUM_SUBCORES,
    )

    @functools.partial(
        pl.kernel,
        out_type=jax.ShapeDtypeStruct((batch, hist, d), jnp.float32),
        mesh=mesh,
        compiler_params=pltpu.CompilerParams(use_tc_tiling_on_sc=True),
        scratch_types=[
            pltpu.VMEM((bp_worker, hist), jnp.int32),
            pltpu.VMEM((_NBUF, _NB, hist, d), jnp.float32),
            pltpu.SemaphoreType.DMA((_NBUF,)),
            pltpu.SemaphoreType.DMA((_NBUF,)),
        ],
    )
    def gather_kernel(idx_hbm, table_hbm, out_hbm, idx_v, bufs, gsem, ssem):
        wid = lax.axis_index("s") * _NUM_CORES + lax.axis_index("c")
        batch0 = wid * bp_worker
        pltpu.sync_copy(idx_hbm.at[pl.ds(batch0, bp_worker)], idx_v)

        def step(g, carry):
            sbase = g * (_NBUF * _NB)
            for b in range(_NBUF):

                @pl.when(g > 0)
                def _():
                    pltpu.make_async_copy(
                        bufs.at[b], out_hbm.at[pl.ds(batch0, _NB)], ssem.at[b]
                    ).wait()

                for i in range(_NB):
                    row = sbase + b * _NB + i
                    pltpu.make_async_copy(
                        table_hbm.at[idx_v.at[row]], bufs.at[b, i], gsem.at[b]
                    ).start()
            for b in range(_NBUF):
                for i in range(_NB):
                    row = sbase + b * _NB + i
                    pltpu.make_async_copy(
                        table_hbm.at[idx_v.at[row]], bufs.at[b, i], gsem.at[b]
                    ).wait()
                pltpu.make_async_copy(
                    bufs.at[b],
                    out_hbm.at[pl.ds(batch0 + sbase + b * _NB, _NB)],
                    ssem.at[b],
                ).start()
            return carry

        lax.fori_loop(0, n_steps, step, 0)
        for b in range(_NBUF):
            pltpu.make_async_copy(
                bufs.at[b], out_hbm.at[pl.ds(batch0, _NB)], ssem.at[b]
            ).wait()

    return gather_kernel


def kernel(input, table):
    b, h = input.shape
    v, d = table.shape
    assert b % (_NUM_WORKERS * _NBUF * _NB) == 0
    idx = input.astype(jnp.int32)
    return _make_gather(b, h, d)(idx, table)
